# Initial kernel scaffold; baseline (speedup 1.0000x reference)
import jax
import jax.numpy as jnp
from jax.experimental import pallas as pl


def kernel(fused_abundances):
    raise NotImplementedError("write your pallas kernel here")


# speedup vs baseline: 45.9533x; 45.9533x over previous
import functools
import math

import jax
import jax.numpy as jnp
from jax import lax
from jax.experimental import pallas as pl
from jax.experimental.pallas import tpu as pltpu
from jax.experimental.pallas import tpu_sc as plsc

_B, _P, _H, _W = 8, 16, 512, 512
_K = 1000
_NPIX = _H * _W
_NPLANES = _B * _P
_NC, _NS, _L = 2, 16, 16
_NWORKERS = _NC * _NS
_PPW = _NPLANES // _NWORKERS
_PERIOD = 2000
_NSTRIPS = _PERIOD // _L
_G = 7
_CH = _G * _PERIOD
_NFULL = _NPIX // _CH
_NPAIR = _NFULL // 2
_REM = _NPIX - _NFULL * _CH
_REM_G = _REM // _PERIOD
_TAIL = _REM - _REM_G * _PERIOD
_TAILV = _TAIL // _L
_REM_OFF = _NFULL * _CH
_E = math.e
_CA = 0.25 / (_E + _K - 1.0)
_CB = 0.5 * (_E - 1.0) / (_E + _K - 1.0)

_mesh = plsc.VectorSubcoreMesh(
    core_axis_name="c", subcore_axis_name="s",
    num_cores=_NC, num_subcores=_NS)


@functools.partial(
    pl.kernel,
    out_type=jax.ShapeDtypeStruct((_NPLANES * _NPIX,), jnp.float32),
    mesh=_mesh,
    compiler_params=pltpu.CompilerParams(use_tc_tiling_on_sc=False,
                                         needs_layout_passes=False),
    scratch_types=[
        pltpu.VMEM((_CH,), jnp.float32),
        pltpu.VMEM((_CH,), jnp.float32),
        pltpu.VMEM((_CH,), jnp.float32),
        pltpu.VMEM((_CH,), jnp.float32),
        pltpu.VMEM((_REM,), jnp.float32),
        pltpu.VMEM((_REM,), jnp.float32),
        pltpu.VMEM((_PERIOD,), jnp.float32),
        pltpu.VMEM((_PERIOD,), jnp.float32),
        pltpu.SemaphoreType.DMA,
        pltpu.SemaphoreType.DMA,
        pltpu.SemaphoreType.DMA,
        pltpu.SemaphoreType.DMA,
        pltpu.SemaphoreType.DMA,
        pltpu.SemaphoreType.DMA,
    ],
)
def _sc_smooth(x_hbm, out_hbm, ib0, ib1, ob0, ob1, rin, rout, sums, g2,
               si0, si1, so0, so1, sri, sro):
    wid = lax.axis_index("s") * _NC + lax.axis_index("c")
    ibufs = (ib0, ib1)
    obufs = (ob0, ob1)
    sis = (si0, si1)
    sos = (so0, so1)

    def start_in(plane, c, buf, sem, size):
        pltpu.make_async_copy(
            x_hbm.at[pl.ds(plane * _NPIX + c * _CH, size)], buf, sem).start()

    def wait_dma(buf, sem):
        pltpu.make_async_copy(x_hbm.at[pl.ds(0, buf.shape[0])],
                              buf, sem).wait()

    def start_out(plane, c, buf, sem, size):
        pltpu.make_async_copy(
            buf, out_hbm.at[pl.ds(plane * _NPIX + c * _CH, size)], sem).start()

    def accum(buf, ngroups):
        def strip(j, carry):
            off = j * _L
            acc = sums[pl.ds(off, _L)]
            for gg in range(ngroups):
                acc = acc + buf[pl.ds(gg * _PERIOD + off, _L)]
            sums[pl.ds(off, _L)] = acc
            return carry
        lax.fori_loop(0, _NSTRIPS, strip, 0, unroll=False)

    def combine(src, dst, ngroups):
        def strip(j, carry):
            off = j * _L
            gv = g2[pl.ds(off, _L)]
            for gg in range(ngroups):
                o = gg * _PERIOD + off
                dst[pl.ds(o, _L)] = src[pl.ds(o, _L)] * 0.5 + gv
            return carry
        lax.fori_loop(0, _NSTRIPS, strip, 0, unroll=False)

    def fold_vals(j):
        idx = j * _L + lax.iota(jnp.int32, _L)
        k = lax.rem(idx, jnp.int32(_K))
        partner = lax.rem(idx + jnp.int32(_K), jnp.int32(_PERIOD))
        sf = sums[pl.ds(j * _L, _L)] + plsc.load_gather(sums, [partner])
        cnt = jnp.where(k < _TAIL,
                        jnp.full((_L,), 263.0, jnp.float32),
                        jnp.full((_L,), 262.0, jnp.float32))
        return sf / cnt

    def run_plane(pp, carry):
        plane = wid * _PPW + pp

        def zero_strip(j, c):
            sums[pl.ds(j * _L, _L)] = jnp.zeros((_L,), jnp.float32)
            return c
        lax.fori_loop(0, _NSTRIPS, zero_strip, 0, unroll=False)

        start_in(plane, 0, ib0, si0, _CH)
        start_in(plane, 1, ib1, si1, _CH)
        pltpu.make_async_copy(
            x_hbm.at[pl.ds(plane * _NPIX + _REM_OFF, _REM)], rin, sri).start()

        def a_pair(cc, c):
            for par in range(2):
                wait_dma(ibufs[par], sis[par])
                accum(ibufs[par], _G)

                @pl.when(cc < _NPAIR - 1)
                def _():
                    start_in(plane, 2 * cc + 2 + par, ibufs[par],
                             sis[par], _CH)
            return c
        lax.fori_loop(0, _NPAIR, a_pair, 0, unroll=False)

        wait_dma(rin, sri)
        accum(rin, _REM_G)
        for j in range(_TAILV):
            off = j * _L
            sums[pl.ds(off, _L)] = (
                sums[pl.ds(off, _L)]
                + rin[pl.ds(_REM_G * _PERIOD + off, _L)])

        def s_body(j, acc):
            return acc + fold_vals(j)
        accv = lax.fori_loop(0, _NSTRIPS, s_body,
                             jnp.zeros((_L,), jnp.float32), unroll=False)
        base = jnp.full((_L,), _CA * jnp.sum(accv), jnp.float32)

        def g_body(j, c):
            g2[pl.ds(j * _L, _L)] = base + _CB * fold_vals(j)
            return c
        lax.fori_loop(0, _NSTRIPS, g_body, 0, unroll=False)

        start_in(plane, 0, ib0, si0, _CH)
        start_in(plane, 1, ib1, si1, _CH)
        pltpu.make_async_copy(
            x_hbm.at[pl.ds(plane * _NPIX + _REM_OFF, _REM)], rin, sri).start()

        def b_pair(cc, c):
            for par in range(2):
                wait_dma(ibufs[par], sis[par])

                @pl.when(cc > 0)
                def _():
                    pltpu.make_async_copy(
                        obufs[par],
                        out_hbm.at[pl.ds(plane * _NPIX, _CH)],
                        sos[par]).wait()
                combine(ibufs[par], obufs[par], _G)
                start_out(plane, 2 * cc + par, obufs[par], sos[par], _CH)

                @pl.when(cc < _NPAIR - 1)
                def _():
                    start_in(plane, 2 * cc + 2 + par, ibufs[par],
                             sis[par], _CH)
            return c
        lax.fori_loop(0, _NPAIR, b_pair, 0, unroll=False)

        wait_dma(rin, sri)
        combine(rin, rout, _REM_G)
        for j in range(_TAILV):
            off = j * _L
            o = _REM_G * _PERIOD + off
            rout[pl.ds(o, _L)] = rin[pl.ds(o, _L)] * 0.5 + g2[pl.ds(off, _L)]
        pltpu.make_async_copy(
            rout, out_hbm.at[pl.ds(plane * _NPIX + _REM_OFF, _REM)], sro).start()

        for par in range(2):
            pltpu.make_async_copy(
                obufs[par], out_hbm.at[pl.ds(plane * _NPIX, _CH)],
                sos[par]).wait()
        pltpu.make_async_copy(
            rout, out_hbm.at[pl.ds(plane * _NPIX + _REM_OFF, _REM)], sro).wait()
        return carry

    lax.fori_loop(0, _PPW, run_plane, 0, unroll=False)


def kernel(fused_abundances):
    x = fused_abundances.reshape(_NPLANES * _NPIX)
    out = _sc_smooth(x)
    return out.reshape(_B, _P, _H, _W)


# speedup vs baseline: 51.0045x; 1.1099x over previous
import functools
import math

import jax
import jax.numpy as jnp
from jax import lax
from jax.experimental import pallas as pl
from jax.experimental.pallas import tpu as pltpu
from jax.experimental.pallas import tpu_sc as plsc

_B, _P, _H, _W = 8, 16, 512, 512
_K = 1000
_NPIX = _H * _W
_NPLANES = _B * _P
_NC, _NS, _L = 2, 16, 16
_NWORKERS = _NC * _NS
_PPW = _NPLANES // _NWORKERS
_PERIOD = 2000
_NSTRIPS = _PERIOD // _L
_G = 7
_CH = _G * _PERIOD
_NFULL = _NPIX // _CH
_NPAIR = _NFULL // 2
_REM = _NPIX - _NFULL * _CH
_REM_G = _REM // _PERIOD
_TAIL = _REM - _REM_G * _PERIOD
_TAILV = _TAIL // _L
_REM_OFF = _NFULL * _CH
_E = math.e
_CA = 0.25 / (_E + _K - 1.0)
_CB = 0.5 * (_E - 1.0) / (_E + _K - 1.0)

_mesh = plsc.VectorSubcoreMesh(
    core_axis_name="c", subcore_axis_name="s",
    num_cores=_NC, num_subcores=_NS)


@functools.partial(
    pl.kernel,
    out_type=jax.ShapeDtypeStruct((_NPLANES * _NPIX,), jnp.float32),
    mesh=_mesh,
    compiler_params=pltpu.CompilerParams(use_tc_tiling_on_sc=False,
                                         needs_layout_passes=False),
    scratch_types=[
        pltpu.VMEM((_CH,), jnp.float32),
        pltpu.VMEM((_CH,), jnp.float32),
        pltpu.VMEM((_CH,), jnp.float32),
        pltpu.VMEM((_CH,), jnp.float32),
        pltpu.VMEM((_REM,), jnp.float32),
        pltpu.VMEM((_REM,), jnp.float32),
        pltpu.VMEM((_PERIOD,), jnp.float32),
        pltpu.VMEM((_PERIOD,), jnp.float32),
        pltpu.SemaphoreType.DMA,
        pltpu.SemaphoreType.DMA,
        pltpu.SemaphoreType.DMA,
        pltpu.SemaphoreType.DMA,
        pltpu.SemaphoreType.DMA,
        pltpu.SemaphoreType.DMA,
    ],
)
def _sc_smooth(x_hbm, out_hbm, ib0, ib1, ob0, ob1, rin, rout, sums, g2,
               si0, si1, so0, so1, sri, sro):
    wid = lax.axis_index("s") * _NC + lax.axis_index("c")
    ibufs = (ib0, ib1)
    obufs = (ob0, ob1)
    sis = (si0, si1)
    sos = (so0, so1)

    def start_in(plane, c, buf, sem, size):
        pltpu.make_async_copy(
            x_hbm.at[pl.ds(plane * _NPIX + c * _CH, size)], buf, sem).start()

    def wait_dma(buf, sem):
        pltpu.make_async_copy(x_hbm.at[pl.ds(0, buf.shape[0])],
                              buf, sem).wait()

    def start_out(plane, c, buf, sem, size):
        pltpu.make_async_copy(
            buf, out_hbm.at[pl.ds(plane * _NPIX + c * _CH, size)], sem).start()

    def accum(buf, ngroups):
        @plsc.parallel_loop(0, _PERIOD, _L, unroll=4)
        def _(off):
            acc = sums[pl.ds(off, _L)]
            for gg in range(ngroups):
                acc = acc + buf[pl.ds(gg * _PERIOD + off, _L)]
            sums[pl.ds(off, _L)] = acc

    def combine(sbuf, dst, ngroups):
        @plsc.parallel_loop(0, _PERIOD, _L, unroll=4)
        def _(off):
            gv = g2[pl.ds(off, _L)]
            for gg in range(ngroups):
                o = gg * _PERIOD + off
                dst[pl.ds(o, _L)] = sbuf[pl.ds(o, _L)] * 0.5 + gv

    def fold_vals(off):
        idx = off + lax.iota(jnp.int32, _L)
        k = lax.rem(idx, jnp.int32(_K))
        partner = lax.rem(idx + jnp.int32(_K), jnp.int32(_PERIOD))
        sf = sums[pl.ds(off, _L)] + plsc.load_gather(sums, [partner])
        cnt = jnp.where(k < _TAIL,
                        jnp.full((_L,), 263.0, jnp.float32),
                        jnp.full((_L,), 262.0, jnp.float32))
        return sf / cnt

    def run_plane(pp, carry):
        plane = wid * _PPW + pp

        @plsc.parallel_loop(0, _PERIOD, _L, unroll=4)
        def _(off):
            sums[pl.ds(off, _L)] = jnp.zeros((_L,), jnp.float32)

        start_in(plane, 0, ib0, si0, _CH)
        start_in(plane, 1, ib1, si1, _CH)
        pltpu.make_async_copy(
            x_hbm.at[pl.ds(plane * _NPIX + _REM_OFF, _REM)], rin, sri).start()

        def a_pair(cc, c):
            for par in range(2):
                wait_dma(ibufs[par], sis[par])
                accum(ibufs[par], _G)

                @pl.when(cc < _NPAIR - 1)
                def _():
                    start_in(plane, 2 * cc + 2 + par, ibufs[par],
                             sis[par], _CH)
            return c
        lax.fori_loop(0, _NPAIR, a_pair, 0, unroll=False)

        wait_dma(rin, sri)
        accum(rin, _REM_G)
        for j in range(_TAILV):
            off = j * _L
            sums[pl.ds(off, _L)] = (
                sums[pl.ds(off, _L)]
                + rin[pl.ds(_REM_G * _PERIOD + off, _L)])

        @plsc.parallel_loop(0, _PERIOD, _L, unroll=2,
                            carry=jnp.zeros((_L,), jnp.float32))
        def accv(off, acc):
            return acc + fold_vals(off)
        base = jnp.full((_L,), _CA * jnp.sum(accv), jnp.float32)

        @plsc.parallel_loop(0, _PERIOD, _L, unroll=2)
        def _(off):
            g2[pl.ds(off, _L)] = base + _CB * fold_vals(off)

        start_in(plane, 0, ib0, si0, _CH)
        start_in(plane, 1, ib1, si1, _CH)
        pltpu.make_async_copy(
            x_hbm.at[pl.ds(plane * _NPIX + _REM_OFF, _REM)], rin, sri).start()

        def b_pair(cc, c):
            for par in range(2):
                wait_dma(ibufs[par], sis[par])

                @pl.when(cc > 0)
                def _():
                    pltpu.make_async_copy(
                        obufs[par],
                        out_hbm.at[pl.ds(plane * _NPIX, _CH)],
                        sos[par]).wait()
                combine(ibufs[par], obufs[par], _G)
                start_out(plane, 2 * cc + par, obufs[par], sos[par], _CH)

                @pl.when(cc < _NPAIR - 1)
                def _():
                    start_in(plane, 2 * cc + 2 + par, ibufs[par],
                             sis[par], _CH)
            return c
        lax.fori_loop(0, _NPAIR, b_pair, 0, unroll=False)

        wait_dma(rin, sri)
        combine(rin, rout, _REM_G)
        for j in range(_TAILV):
            off = j * _L
            o = _REM_G * _PERIOD + off
            rout[pl.ds(o, _L)] = rin[pl.ds(o, _L)] * 0.5 + g2[pl.ds(off, _L)]
        pltpu.make_async_copy(
            rout, out_hbm.at[pl.ds(plane * _NPIX + _REM_OFF, _REM)], sro).start()

        for par in range(2):
            pltpu.make_async_copy(
                obufs[par], out_hbm.at[pl.ds(plane * _NPIX, _CH)],
                sos[par]).wait()
        pltpu.make_async_copy(
            rout, out_hbm.at[pl.ds(plane * _NPIX + _REM_OFF, _REM)], sro).wait()
        return carry

    lax.fori_loop(0, _PPW, run_plane, 0, unroll=False)


def kernel(fused_abundances):
    x = fused_abundances.reshape(_NPLANES * _NPIX)
    out = _sc_smooth(x)
    return out.reshape(_B, _P, _H, _W)


# speedup vs baseline: 64.4758x; 1.2641x over previous
import functools
import math

import jax
import jax.numpy as jnp
from jax import lax
from jax.experimental import pallas as pl
from jax.experimental.pallas import tpu as pltpu
from jax.experimental.pallas import tpu_sc as plsc

_B, _P, _H, _W = 8, 16, 512, 512
_K = 1000
_NPIX = _H * _W
_NPLANES = _B * _P
_NROWS = _NPLANES * _H
_NC, _NS, _L = 2, 16, 16
_NWORKERS = _NC * _NS
_PPW = _NPLANES // _NWORKERS
_PERIOD = 2000
_PU = _PERIOD // _L
_RCH = 32
_CH = _RCH * _W
_NCH = _NPIX // _CH
_NPAIR = _NCH // 2
_GF = _CH // _PERIOD
_TAIL = _CH - _GF * _PERIOD
_E = math.e
_CA = 0.25 / (_E + _K - 1.0)
_CB = 0.5 * (_E - 1.0) / (_E + _K - 1.0)
_B0P = (_NPIX // _L) % _PU
_B0C = (_CH // _L) % _PU

_mesh = plsc.VectorSubcoreMesh(
    core_axis_name="c", subcore_axis_name="s",
    num_cores=_NC, num_subcores=_NS)


@functools.partial(
    pl.kernel,
    out_type=jax.ShapeDtypeStruct((_NROWS, _W), jnp.float32),
    mesh=_mesh,
    compiler_params=pltpu.CompilerParams(use_tc_tiling_on_sc=True,
                                         needs_layout_passes=False),
    scratch_types=[
        pltpu.VMEM((_RCH, _W), jnp.float32),
        pltpu.VMEM((_RCH, _W), jnp.float32),
        pltpu.VMEM((_RCH, _W), jnp.float32),
        pltpu.VMEM((_RCH, _W), jnp.float32),
        pltpu.VMEM((_PERIOD,), jnp.float32),
        pltpu.VMEM((_PERIOD,), jnp.float32),
        pltpu.SemaphoreType.DMA,
        pltpu.SemaphoreType.DMA,
        pltpu.SemaphoreType.DMA,
        pltpu.SemaphoreType.DMA,
    ],
)
def _sc_smooth(x_hbm, out_hbm, ib0, ib1, ob0, ob1, sums, g2,
               si0, si1, so0, so1):
    wid = lax.axis_index("s") * _NC + lax.axis_index("c")
    ibufs = (ib0, ib1)
    obufs = (ob0, ob1)
    sis = (si0, si1)
    sos = (so0, so1)

    def chunk_rows(plane, k):
        return plane * _H + k * _RCH

    def start_in(plane, k, buf, sem):
        pltpu.make_async_copy(
            x_hbm.at[pl.ds(chunk_rows(plane, k), _RCH), :], buf, sem).start()

    def wait_dma(buf, sem):
        pltpu.make_async_copy(x_hbm.at[pl.ds(0, _RCH), :], buf, sem).wait()

    def start_out(plane, k, buf, sem):
        pltpu.make_async_copy(
            buf, out_hbm.at[pl.ds(chunk_rows(plane, k), _RCH), :], sem).start()

    def phase_units(plane, k):
        return lax.rem(plane * _B0P + k * _B0C, jnp.int32(_PU))

    def rc(pos):
        return lax.shift_right_logical(pos, 9), lax.bitwise_and(pos, _W - 1)

    def accum(buf, b0u):
        @plsc.parallel_loop(0, _PU, 1, unroll=2)
        def _(tu):
            t = tu * _L
            u = b0u + tu
            u = jnp.where(u >= _PU, u - _PU, u)
            off = u * _L
            acc = jnp.zeros((_L,), jnp.float32)
            for g in range(_GF):
                r, c = rc(t + _PERIOD * g)
                acc = acc + buf[r, pl.ds(c, _L)]

            @pl.when(tu < _TAIL // _L)
            def _():
                r, c = rc(t + _GF * _PERIOD)
                sums[pl.ds(off, _L)] = (
                    sums[pl.ds(off, _L)] + acc + buf[r, pl.ds(c, _L)])

            @pl.when(tu >= _TAIL // _L)
            def _():
                sums[pl.ds(off, _L)] = sums[pl.ds(off, _L)] + acc

    def combine(bufin, bufout, b0u):
        @plsc.parallel_loop(0, _PU, 1, unroll=2)
        def _(tu):
            t = tu * _L
            u = b0u + tu
            u = jnp.where(u >= _PU, u - _PU, u)
            off = u * _L
            gv = g2[pl.ds(off, _L)]
            for g in range(_GF):
                r, c = rc(t + _PERIOD * g)
                bufout[r, pl.ds(c, _L)] = bufin[r, pl.ds(c, _L)] * 0.5 + gv

            @pl.when(tu < _TAIL // _L)
            def _():
                r, c = rc(t + _GF * _PERIOD)
                bufout[r, pl.ds(c, _L)] = bufin[r, pl.ds(c, _L)] * 0.5 + gv

    def fold_vals(off):
        idx = off + lax.iota(jnp.int32, _L)
        kk = lax.rem(idx, jnp.int32(_K))
        partner = lax.rem(idx + jnp.int32(_K), jnp.int32(_PERIOD))
        sf = sums[pl.ds(off, _L)] + plsc.load_gather(sums, [partner])
        cnt = jnp.where(kk < 144,
                        jnp.full((_L,), 263.0, jnp.float32),
                        jnp.full((_L,), 262.0, jnp.float32))
        return sf / cnt

    def run_plane(pp, carry):
        plane = wid * _PPW + pp

        @plsc.parallel_loop(0, _PERIOD, _L, unroll=4)
        def _(off):
            sums[pl.ds(off, _L)] = jnp.zeros((_L,), jnp.float32)

        start_in(plane, 0, ib0, si0)
        start_in(plane, 1, ib1, si1)

        def a_pair(cc, c):
            for par in range(2):
                k = 2 * cc + par
                wait_dma(ibufs[par], sis[par])
                accum(ibufs[par], phase_units(plane, k))

                @pl.when(cc < _NPAIR - 1)
                def _():
                    start_in(plane, k + 2, ibufs[par], sis[par])
            return c
        lax.fori_loop(0, _NPAIR, a_pair, 0, unroll=False)

        @plsc.parallel_loop(0, _PERIOD, _L, unroll=2,
                            carry=jnp.zeros((_L,), jnp.float32))
        def accv(off, acc):
            return acc + fold_vals(off)
        base = jnp.full((_L,), _CA * jnp.sum(accv), jnp.float32)

        @plsc.parallel_loop(0, _PERIOD, _L, unroll=2)
        def _(off):
            g2[pl.ds(off, _L)] = base + _CB * fold_vals(off)

        start_in(plane, 0, ib0, si0)
        start_in(plane, 1, ib1, si1)

        def b_pair(cc, c):
            for par in range(2):
                k = 2 * cc + par
                wait_dma(ibufs[par], sis[par])

                @pl.when(cc > 0)
                def _():
                    pltpu.make_async_copy(
                        obufs[par],
                        out_hbm.at[pl.ds(0, _RCH), :],
                        sos[par]).wait()
                combine(ibufs[par], obufs[par], phase_units(plane, k))
                start_out(plane, k, obufs[par], sos[par])

                @pl.when(cc < _NPAIR - 1)
                def _():
                    start_in(plane, k + 2, ibufs[par], sis[par])
            return c
        lax.fori_loop(0, _NPAIR, b_pair, 0, unroll=False)

        for par in range(2):
            pltpu.make_async_copy(
                obufs[par], out_hbm.at[pl.ds(0, _RCH), :], sos[par]).wait()
        return carry

    lax.fori_loop(0, _PPW, run_plane, 0, unroll=False)


def kernel(fused_abundances):
    x = fused_abundances.reshape(_NROWS, _W)
    out = _sc_smooth(x)
    return out.reshape(_B, _P, _H, _W)


# speedup vs baseline: 74.4823x; 1.1552x over previous
import functools
import math

import jax
import jax.numpy as jnp
from jax import lax
from jax.experimental import pallas as pl
from jax.experimental.pallas import tpu as pltpu
from jax.experimental.pallas import tpu_sc as plsc

_B, _P, _H, _W = 8, 16, 512, 512
_K = 1000
_NPIX = _H * _W
_NPLANES = _B * _P
_NROWS = _NPLANES * _H
_NC, _NS, _L = 2, 16, 16
_NWORKERS = _NC * _NS
_PPW = _NPLANES // _NWORKERS
_PERIOD = 2000
_PU = _PERIOD // _L
_RCH = 32
_CH = _RCH * _W
_NCH = _NPIX // _CH
_NPAIR = _NCH // 2
_GF = _CH // _PERIOD
_TAIL = _CH - _GF * _PERIOD
_E = math.e
_CA = 0.25 / (_E + _K - 1.0)
_CB = 0.5 * (_E - 1.0) / (_E + _K - 1.0)
_B0P = (_NPIX // _L) % _PU
_B0C = (_CH // _L) % _PU

_mesh = plsc.VectorSubcoreMesh(
    core_axis_name="c", subcore_axis_name="s",
    num_cores=_NC, num_subcores=_NS)


@functools.partial(
    pl.kernel,
    out_type=jax.ShapeDtypeStruct((_NROWS, _W), jnp.float32),
    mesh=_mesh,
    compiler_params=pltpu.CompilerParams(use_tc_tiling_on_sc=True,
                                         needs_layout_passes=False),
    scratch_types=[
        pltpu.VMEM((_RCH, _W), jnp.float32),
        pltpu.VMEM((_RCH, _W), jnp.float32),
        pltpu.VMEM((_RCH, _W), jnp.float32),
        pltpu.VMEM((_RCH, _W), jnp.float32),
        pltpu.VMEM((_PERIOD,), jnp.float32),
        pltpu.VMEM((_PERIOD,), jnp.float32),
        pltpu.SemaphoreType.DMA,
        pltpu.SemaphoreType.DMA,
        pltpu.SemaphoreType.DMA,
        pltpu.SemaphoreType.DMA,
    ],
)
def _sc_smooth(x_hbm, out_hbm, ib0, ib1, ob0, ob1, sums, g2,
               si0, si1, so0, so1):
    wid = lax.axis_index("s") * _NC + lax.axis_index("c")
    ibufs = (ib0, ib1)
    obufs = (ob0, ob1)
    sis = (si0, si1)
    sos = (so0, so1)

    def chunk_rows(plane, k):
        return plane * _H + k * _RCH

    def start_in(plane, k, buf, sem):
        pltpu.make_async_copy(
            x_hbm.at[pl.ds(chunk_rows(plane, k), _RCH), :], buf, sem).start()

    def wait_dma(buf, sem):
        pltpu.make_async_copy(x_hbm.at[pl.ds(0, _RCH), :], buf, sem).wait()

    def start_out(plane, k, buf, sem):
        pltpu.make_async_copy(
            buf, out_hbm.at[pl.ds(chunk_rows(plane, k), _RCH), :], sem).start()

    def phase_units(plane, k):
        return lax.rem(plane * _B0P + k * _B0C, jnp.int32(_PU))

    def rc(pos):
        return lax.shift_right_logical(pos, 9), lax.bitwise_and(pos, _W - 1)

    _TU = _TAIL // _L

    def accum(buf, b0u):
        def body(ngroups):
            def f(tu):
                t = tu * _L
                u = b0u + tu
                u = jnp.where(u >= _PU, u - _PU, u)
                off = u * _L
                r0, c0 = rc(t)
                acc0 = buf[r0, pl.ds(c0, _L)]
                r1, c1 = rc(t + _PERIOD)
                acc1 = buf[r1, pl.ds(c1, _L)]
                for g in range(2, ngroups):
                    r, c = rc(t + _PERIOD * g)
                    if g % 2 == 0:
                        acc0 = acc0 + buf[r, pl.ds(c, _L)]
                    else:
                        acc1 = acc1 + buf[r, pl.ds(c, _L)]
                sums[pl.ds(off, _L)] = sums[pl.ds(off, _L)] + (acc0 + acc1)
            return f
        plsc.parallel_loop(0, _TU, 1, unroll=4)(body(_GF + 1))
        plsc.parallel_loop(_TU, _PU, 1, unroll=4)(body(_GF))

    def combine(bufin, bufout, b0u):
        def body(ngroups):
            def f(tu):
                t = tu * _L
                u = b0u + tu
                u = jnp.where(u >= _PU, u - _PU, u)
                off = u * _L
                gv = g2[pl.ds(off, _L)]
                for g in range(ngroups):
                    r, c = rc(t + _PERIOD * g)
                    bufout[r, pl.ds(c, _L)] = (
                        bufin[r, pl.ds(c, _L)] * 0.5 + gv)
            return f
        plsc.parallel_loop(0, _TU, 1, unroll=4)(body(_GF + 1))
        plsc.parallel_loop(_TU, _PU, 1, unroll=4)(body(_GF))

    def fold_vals(off):
        idx = off + lax.iota(jnp.int32, _L)
        kk = lax.rem(idx, jnp.int32(_K))
        partner = lax.rem(idx + jnp.int32(_K), jnp.int32(_PERIOD))
        sf = sums[pl.ds(off, _L)] + plsc.load_gather(sums, [partner])
        cnt = jnp.where(kk < 144,
                        jnp.full((_L,), 263.0, jnp.float32),
                        jnp.full((_L,), 262.0, jnp.float32))
        return sf / cnt

    def run_plane(pp, carry):
        plane = wid * _PPW + pp

        @plsc.parallel_loop(0, _PERIOD, _L, unroll=4)
        def _(off):
            sums[pl.ds(off, _L)] = jnp.zeros((_L,), jnp.float32)

        start_in(plane, 0, ib0, si0)
        start_in(plane, 1, ib1, si1)

        def a_pair(cc, c):
            for par in range(2):
                k = 2 * cc + par
                wait_dma(ibufs[par], sis[par])
                accum(ibufs[par], phase_units(plane, k))

                @pl.when(cc < _NPAIR - 1)
                def _():
                    start_in(plane, k + 2, ibufs[par], sis[par])
            return c
        lax.fori_loop(0, _NPAIR, a_pair, 0, unroll=False)

        @plsc.parallel_loop(0, _PERIOD, _L, unroll=2,
                            carry=jnp.zeros((_L,), jnp.float32))
        def accv(off, acc):
            return acc + fold_vals(off)
        base = jnp.full((_L,), _CA * jnp.sum(accv), jnp.float32)

        @plsc.parallel_loop(0, _PERIOD, _L, unroll=2)
        def _(off):
            g2[pl.ds(off, _L)] = base + _CB * fold_vals(off)

        start_in(plane, 0, ib0, si0)
        start_in(plane, 1, ib1, si1)

        def b_pair(cc, c):
            for par in range(2):
                k = 2 * cc + par
                wait_dma(ibufs[par], sis[par])

                @pl.when(cc > 0)
                def _():
                    pltpu.make_async_copy(
                        obufs[par],
                        out_hbm.at[pl.ds(0, _RCH), :],
                        sos[par]).wait()
                combine(ibufs[par], obufs[par], phase_units(plane, k))
                start_out(plane, k, obufs[par], sos[par])

                @pl.when(cc < _NPAIR - 1)
                def _():
                    start_in(plane, k + 2, ibufs[par], sis[par])
            return c
        lax.fori_loop(0, _NPAIR, b_pair, 0, unroll=False)

        for par in range(2):
            pltpu.make_async_copy(
                obufs[par], out_hbm.at[pl.ds(0, _RCH), :], sos[par]).wait()
        return carry

    lax.fori_loop(0, _PPW, run_plane, 0, unroll=False)


def kernel(fused_abundances):
    x = fused_abundances.reshape(_NROWS, _W)
    out = _sc_smooth(x)
    return out.reshape(_B, _P, _H, _W)
